# baseline (speedup 1.0000x reference)
import functools

import jax
import jax.numpy as jnp
from jax import lax
from jax.experimental import pallas as pl
from jax.experimental.pallas import tpu as pltpu
from jax.experimental.pallas import tpu_sc as plsc

B = 16384
IN_C = 67
OUT_C = 73
NW = 32
CPW = B // NW
L = 16


def _sc_body(elev_t_hbm, tab_hbm, out_t_hbm, in_v, buf, tab_v):
    wid = lax.axis_index("s") * 2 + lax.axis_index("c")
    cols = pl.ds(wid * CPW, CPW)

    pltpu.sync_copy(elev_t_hbm.at[:, cols], in_v)
    pltpu.sync_copy(tab_hbm, tab_v)

    @plsc.parallel_loop(0, CPW // L, unroll=2)
    def chunk(j):
        sl = pl.ds(j * L, L)
        idx8 = in_v[18, sl].astype(jnp.int32) * 8
        for c in range(16):
            buf[c, sl] = in_v[c, sl]
        buf[16, sl] = in_v[17, sl]
        for c in range(17, 65):
            buf[c, sl] = in_v[c + 2, sl]
        for e in range(8):
            buf[65 + e, sl] = plsc.load_gather(tab_v, [idx8 + e])

    pltpu.sync_copy(buf, out_t_hbm.at[:, cols])


_sc_kernel = functools.partial(
    pl.kernel,
    out_type=jax.ShapeDtypeStruct((OUT_C, B), jnp.float32),
    mesh=plsc.VectorSubcoreMesh(core_axis_name="c", subcore_axis_name="s"),
    compiler_params=pltpu.CompilerParams(
        needs_layout_passes=False, use_tc_tiling_on_sc=True,
        skip_device_barrier=True),
    scratch_types=[
        pltpu.VMEM((IN_C, CPW), jnp.float32),
        pltpu.VMEM((OUT_C, CPW), jnp.float32),
        pltpu.VMEM((32,), jnp.float32),
    ],
)(_sc_body)


@jax.jit
def kernel(elev_info, door_table, srv_dir_table):
    del srv_dir_table
    out_t = _sc_kernel(elev_info.T, door_table.reshape(-1))
    return out_t.T
